# baseline (speedup 1.0000x reference)
import functools

import jax
import jax.numpy as jnp
from jax import lax
from jax.experimental import pallas as pl
from jax.experimental.pallas import tpu as pltpu
from jax.experimental.pallas import tpu_sc as plsc

D = 256
K = 8192
M = 8192
COMMITMENT_COST = 0.25

MH = M // 2
MT = 2048
NT = 2048
MTE = 1024
NTE = 4096
H = 2048

_SC_CORES = 2
_SC_SUBCORES = 16
_NW = _SC_CORES * _SC_SUBCORES
_BPW = M // _NW
_RPW = H // _NW

_DOT_PRECISION = lax.Precision.DEFAULT


def _dist_body(x_ref, e_ref, d_ref, idx_ref, mv_ref, minval, minidx):
    n = pl.program_id(1)
    x = x_ref[...]
    e = e_ref[...]
    xe = jnp.dot(x, e, preferred_element_type=jnp.float32,
                 precision=_DOT_PRECISION)
    x2 = jnp.sum(x * x, axis=1, keepdims=True)
    e2 = jnp.sum(e * e, axis=0, keepdims=True)
    d = (x2 - 2.0 * xe) + e2
    d_ref[...] = d
    rmin = jnp.min(d, axis=1, keepdims=True)
    col = lax.broadcasted_iota(jnp.int32, d.shape, 1)
    ridx = jnp.min(jnp.where(d == rmin, col, jnp.int32(2**31 - 1)),
                   axis=1, keepdims=True) + n * NT

    @pl.when(n == 0)
    def _():
        minval[...] = rmin
        minidx[...] = ridx

    @pl.when(n != 0)
    def _():
        mv = minval[...]
        better = rmin < mv
        minval[...] = jnp.where(better, rmin, mv)
        minidx[...] = jnp.where(better, ridx, minidx[...])

    @pl.when(n == pl.num_programs(1) - 1)
    def _():
        idx_ref[0] = minidx[...]
        mv_ref[0] = minval[...]


@functools.cache
def _dist1_call():
    return pl.pallas_call(
        _dist_body,
        grid=(MH // MT, K // NT),
        in_specs=[
            pl.BlockSpec((MT, D), lambda m, n: (m, 0)),
            pl.BlockSpec((D, NT), lambda m, n: (0, n)),
        ],
        out_specs=[
            pl.BlockSpec((MT, NT), lambda m, n: (m, n)),
            pl.BlockSpec((1, MT, 1), lambda m, n: (m, 0, 0)),
            pl.BlockSpec((1, MT, 1), lambda m, n: (m, 0, 0)),
        ],
        out_shape=[
            jax.ShapeDtypeStruct((M, K), jnp.float32),
            jax.ShapeDtypeStruct((MH // MT, MT, 1), jnp.int32),
            jax.ShapeDtypeStruct((MH // MT, MT, 1), jnp.float32),
        ],
        scratch_shapes=[
            pltpu.VMEM((MT, 1), jnp.float32),
            pltpu.VMEM((MT, 1), jnp.int32),
        ],
    )


def _dist2_body(x_ref, e_ref, d_in_ref, d_ref, idx_ref, mv_ref, minval,
                minidx):
    del d_in_ref
    _dist_body(x_ref, e_ref, d_ref, idx_ref, mv_ref, minval, minidx)


@functools.cache
def _dist2_call():
    hoff = MH // MT
    return pl.pallas_call(
        _dist2_body,
        grid=(MH // MT, K // NT),
        in_specs=[
            pl.BlockSpec((MT, D), lambda m, n: (m + hoff, 0)),
            pl.BlockSpec((D, NT), lambda m, n: (0, n)),
            pl.BlockSpec(memory_space=pltpu.HBM),
        ],
        out_specs=[
            pl.BlockSpec((MT, NT), lambda m, n: (m + hoff, n)),
            pl.BlockSpec((1, MT, 1), lambda m, n: (m, 0, 0)),
            pl.BlockSpec((1, MT, 1), lambda m, n: (m, 0, 0)),
        ],
        out_shape=[
            jax.ShapeDtypeStruct((M, K), jnp.float32),
            jax.ShapeDtypeStruct((MH // MT, MT, 1), jnp.int32),
            jax.ShapeDtypeStruct((MH // MT, MT, 1), jnp.float32),
        ],
        scratch_shapes=[
            pltpu.VMEM((MT, 1), jnp.float32),
            pltpu.VMEM((MT, 1), jnp.int32),
        ],
        input_output_aliases={2: 0},
    )


def _enc_body(idx_ref, enc_in_ref, enc_ref, cnt_ref, cnt_acc):
    del enc_in_ref
    n = pl.program_id(0)
    m = pl.program_id(1)
    idxv = idx_ref[0]
    col = lax.broadcasted_iota(jnp.int32, (MTE, NTE), 1) + n * NTE
    enc = (col == idxv).astype(jnp.float32)
    enc_ref[...] = enc
    csum = jnp.sum(enc, axis=0, keepdims=True)

    @pl.when(m == 0)
    def _():
        cnt_acc[...] = csum

    @pl.when(m != 0)
    def _():
        cnt_acc[...] += csum

    @pl.when(m == pl.num_programs(1) - 1)
    def _():
        cnt_ref[...] = cnt_acc[...]


@functools.cache
def _enc_call():
    hoff = H // MTE
    return pl.pallas_call(
        _enc_body,
        grid=(K // NTE, (M - H) // MTE),
        in_specs=[
            pl.BlockSpec((1, MTE, 1), lambda n, m: (m + hoff, 0, 0)),
            pl.BlockSpec(memory_space=pltpu.HBM),
        ],
        out_specs=[
            pl.BlockSpec((MTE, NTE), lambda n, m: (m + hoff, n)),
            pl.BlockSpec((1, NTE), lambda n, m: (0, n)),
        ],
        out_shape=[
            jax.ShapeDtypeStruct((M, K), jnp.float32),
            jax.ShapeDtypeStruct((1, K), jnp.float32),
        ],
        scratch_shapes=[
            pltpu.VMEM((1, NTE), jnp.float32),
        ],
        input_output_aliases={1: 0},
    )


def _cnt_body(idx_ref, cnt_ref, cnt_acc):
    n = pl.program_id(0)
    m = pl.program_id(1)
    idxv = idx_ref[0]
    col = lax.broadcasted_iota(jnp.int32, (MTE, NTE), 1) + n * NTE
    csum = jnp.sum((col == idxv).astype(jnp.float32), axis=0, keepdims=True)

    @pl.when(m == 0)
    def _():
        cnt_acc[...] = csum

    @pl.when(m != 0)
    def _():
        cnt_acc[...] += csum

    @pl.when(m == pl.num_programs(1) - 1)
    def _():
        cnt_ref[...] = cnt_acc[...]


@functools.cache
def _cnt_call():
    return pl.pallas_call(
        _cnt_body,
        grid=(K // NTE, H // MTE),
        in_specs=[
            pl.BlockSpec((1, MTE, 1), lambda n, m: (m, 0, 0)),
        ],
        out_specs=[
            pl.BlockSpec((1, NTE), lambda n, m: (0, n)),
        ],
        out_shape=[jax.ShapeDtypeStruct((1, K), jnp.float32)],
        scratch_shapes=[
            pltpu.VMEM((1, NTE), jnp.float32),
        ],
    )


def _fin_body(mv1_ref, mv2_ref, cnt1_ref, cnt2_ref, loss_ref, perp_ref):
    s = jnp.sum(mv1_ref[...]) + jnp.sum(mv2_ref[...])
    loss_ref[0, 0] = s * ((1.0 + COMMITMENT_COST) / (M * D))
    p = (cnt1_ref[...] + cnt2_ref[...]) * (1.0 / M)
    h = jnp.sum(p * jnp.log(p + 1e-10))
    perp_ref[0, 0] = jnp.exp(-h)


@functools.cache
def _fin_call():
    return pl.pallas_call(
        _fin_body,
        in_specs=[
            pl.BlockSpec(memory_space=pltpu.VMEM),
            pl.BlockSpec(memory_space=pltpu.VMEM),
            pl.BlockSpec(memory_space=pltpu.VMEM),
            pl.BlockSpec(memory_space=pltpu.VMEM),
        ],
        out_specs=[
            pl.BlockSpec(memory_space=pltpu.SMEM),
            pl.BlockSpec(memory_space=pltpu.SMEM),
        ],
        out_shape=[
            jax.ShapeDtypeStruct((1, 1), jnp.float32),
            jax.ShapeDtypeStruct((1, 1), jnp.float32),
        ],
    )


@functools.cache
def _sc_enc_call():
    @functools.partial(
        pl.kernel,
        out_type=jax.ShapeDtypeStruct((M, K), jnp.float32),
        mesh=plsc.VectorSubcoreMesh(core_axis_name="c", subcore_axis_name="s"),
        scratch_types=[
            pltpu.VMEM((_RPW,), jnp.int32),
            pltpu.VMEM((8, K), jnp.float32),
        ],
    )
    def _sc_enc(idx_hbm, enc_hbm, idx_v, buf):
        wid = lax.axis_index("s") * _SC_CORES + lax.axis_index("c")
        base = wid * _RPW
        pltpu.sync_copy(idx_hbm.at[pl.ds(base, _RPW)], idx_v)

        zeros16 = jnp.zeros((16,), jnp.float32)

        for r in range(8):
            def _zero(i, carry, _r=r):
                buf[_r, pl.ds(i * 16, 16)] = zeros16
                return carry

            lax.fori_loop(0, K // 16, _zero, jnp.int32(0))

        lane = lax.broadcasted_iota(jnp.int32, (16,), 0)
        ones = jnp.ones((16,), jnp.float32)
        mask_a = lane < 8
        mask_b = lane >= 8
        flat_a = lane * K
        flat_b = (lane - 8) * K

        def _group(g, carry):
            rb = base + g * 16
            pltpu.sync_copy(buf, enc_hbm.at[pl.ds(rb, 8)])
            pltpu.sync_copy(buf, enc_hbm.at[pl.ds(rb + 8, 8)])
            return carry

        lax.fori_loop(0, _RPW // 16, _group, jnp.int32(0))

    return _sc_enc


@functools.cache
def _sc_gather_call():
    @functools.partial(
        pl.kernel,
        out_type=jax.ShapeDtypeStruct((M, D), jnp.float32),
        mesh=plsc.VectorSubcoreMesh(core_axis_name="c", subcore_axis_name="s"),
        scratch_types=[
            pltpu.VMEM((_BPW,), jnp.int32),
            pltpu.VMEM((_BPW, D), jnp.float32),
            pltpu.SemaphoreType.DMA,
        ],
    )
    def _sc_gather(table_hbm, idx_hbm, out_hbm, idx_v, rows_v, sem):
        wid = lax.axis_index("s") * _SC_CORES + lax.axis_index("c")
        base = wid * _BPW
        pltpu.sync_copy(idx_hbm.at[pl.ds(base, _BPW)], idx_v)
        pltpu.async_copy(table_hbm.at[idx_v], rows_v, sem).wait()
        pltpu.sync_copy(rows_v, out_hbm.at[pl.ds(base, _BPW)])

    return _sc_gather


def kernel(inputs, embeddings, is_training):
    x = inputs.reshape(M, D)
    dist1, idx1, mv1 = _dist1_call()(x, embeddings)
    dist, idx2, mv2 = _dist2_call()(x, embeddings, dist1)
    idx_flat = jnp.concatenate([idx1.reshape(MH), idx2.reshape(MH)])
    idx_r = idx_flat.reshape(M // MTE, MTE, 1)

    enc_sc = _sc_enc_call()(idx1.reshape(MH))
    codebook = jnp.swapaxes(embeddings, 0, 1)
    quant = _sc_gather_call()(codebook, idx_flat)

    (cnt_sc,) = _cnt_call()(idx_r)
    enc, cnt_tc = _enc_call()(idx_r, enc_sc)
    loss2, perp2 = _fin_call()(mv1, mv2, cnt_sc, cnt_tc)
    return (
        quant.reshape(inputs.shape),
        loss2.reshape(()),
        perp2.reshape(()),
        enc,
        idx_flat.reshape(inputs.shape[:-1]),
        dist,
    )


# speedup vs baseline: 1.0624x; 1.0624x over previous
import functools

import jax
import jax.numpy as jnp
from jax import lax
from jax.experimental import pallas as pl
from jax.experimental.pallas import tpu as pltpu
from jax.experimental.pallas import tpu_sc as plsc

D = 256
K = 8192
M = 8192
COMMITMENT_COST = 0.25

MT = 2048
NT = 2048
MTE = 1024
NTE = 4096

_SC_CORES = 2
_SC_SUBCORES = 16
_NW = _SC_CORES * _SC_SUBCORES
_BPW = M // _NW

_DOT_PRECISION = lax.Precision.DEFAULT


def _dist_body(x_ref, e_ref, d_ref, idx_ref, mv_ref, minval, minidx, x2s):
    n = pl.program_id(1)
    x = x_ref[...]
    e = e_ref[:, pl.ds(n * NT, NT)]

    @pl.when(n == 0)
    def _():
        x2s[...] = jnp.sum(x * x, axis=1, keepdims=True)

    xe = jnp.dot(x, e, preferred_element_type=jnp.float32,
                 precision=_DOT_PRECISION)
    x2 = x2s[...]
    e2 = jnp.sum(e * e, axis=0, keepdims=True)
    d = (x2 - 2.0 * xe) + e2
    d_ref[...] = d
    rmin = jnp.min(d, axis=1, keepdims=True)
    col = lax.broadcasted_iota(jnp.int32, d.shape, 1)
    ridx = jnp.min(jnp.where(d == rmin, col, jnp.int32(2**31 - 1)),
                   axis=1, keepdims=True) + n * NT

    @pl.when(n == 0)
    def _():
        minval[...] = rmin
        minidx[...] = ridx

    @pl.when(n != 0)
    def _():
        mv = minval[...]
        better = rmin < mv
        minval[...] = jnp.where(better, rmin, mv)
        minidx[...] = jnp.where(better, ridx, minidx[...])

    @pl.when(n == pl.num_programs(1) - 1)
    def _():
        idx_ref[0] = minidx[...]
        mv_ref[0] = minval[...]


@functools.cache
def _dist_call():
    return pl.pallas_call(
        _dist_body,
        grid=(M // MT, K // NT),
        in_specs=[
            pl.BlockSpec((MT, D), lambda m, n: (m, 0)),
            pl.BlockSpec((D, K), lambda m, n: (0, 0)),
        ],
        out_specs=[
            pl.BlockSpec((MT, NT), lambda m, n: (m, n)),
            pl.BlockSpec((1, MT, 1), lambda m, n: (m, 0, 0)),
            pl.BlockSpec((1, MT, 1), lambda m, n: (m, 0, 0)),
        ],
        out_shape=[
            jax.ShapeDtypeStruct((M, K), jnp.float32),
            jax.ShapeDtypeStruct((M // MT, MT, 1), jnp.int32),
            jax.ShapeDtypeStruct((M // MT, MT, 1), jnp.float32),
        ],
        scratch_shapes=[
            pltpu.VMEM((MT, 1), jnp.float32),
            pltpu.VMEM((MT, 1), jnp.int32),
            pltpu.VMEM((MT, 1), jnp.float32),
        ],
    )


def _enc_body(idx_ref, enc_ref, cnt_ref, cnt_acc):
    n = pl.program_id(0)
    m = pl.program_id(1)
    idxv = idx_ref[0]
    col = lax.broadcasted_iota(jnp.int32, (MTE, NTE), 1) + n * NTE
    enc = (col == idxv).astype(jnp.float32)
    enc_ref[...] = enc
    csum = jnp.sum(enc, axis=0, keepdims=True)

    @pl.when(m == 0)
    def _():
        cnt_acc[...] = csum

    @pl.when(m != 0)
    def _():
        cnt_acc[...] += csum

    @pl.when(m == pl.num_programs(1) - 1)
    def _():
        cnt_ref[...] = cnt_acc[...]


@functools.cache
def _enc_call():
    return pl.pallas_call(
        _enc_body,
        grid=(K // NTE, M // MTE),
        in_specs=[
            pl.BlockSpec((1, MTE, 1), lambda n, m: (m, 0, 0)),
        ],
        out_specs=[
            pl.BlockSpec((MTE, NTE), lambda n, m: (m, n)),
            pl.BlockSpec((1, NTE), lambda n, m: (0, n)),
        ],
        out_shape=[
            jax.ShapeDtypeStruct((M, K), jnp.float32),
            jax.ShapeDtypeStruct((1, K), jnp.float32),
        ],
        scratch_shapes=[
            pltpu.VMEM((1, NTE), jnp.float32),
        ],
    )


def _fin_body(mv_ref, cnt_ref, loss_ref, perp_ref):
    s = jnp.sum(mv_ref[...])
    loss_ref[0, 0] = s * ((1.0 + COMMITMENT_COST) / (M * D))
    p = cnt_ref[...] * (1.0 / M)
    h = jnp.sum(p * jnp.log(p + 1e-10))
    perp_ref[0, 0] = jnp.exp(-h)


@functools.cache
def _fin_call():
    return pl.pallas_call(
        _fin_body,
        in_specs=[
            pl.BlockSpec(memory_space=pltpu.VMEM),
            pl.BlockSpec(memory_space=pltpu.VMEM),
        ],
        out_specs=[
            pl.BlockSpec(memory_space=pltpu.SMEM),
            pl.BlockSpec(memory_space=pltpu.SMEM),
        ],
        out_shape=[
            jax.ShapeDtypeStruct((1, 1), jnp.float32),
            jax.ShapeDtypeStruct((1, 1), jnp.float32),
        ],
    )


@functools.cache
def _sc_gather_call():
    @functools.partial(
        pl.kernel,
        out_type=jax.ShapeDtypeStruct((M, D), jnp.float32),
        mesh=plsc.VectorSubcoreMesh(core_axis_name="c", subcore_axis_name="s"),
        scratch_types=[
            pltpu.VMEM((_BPW,), jnp.int32),
            pltpu.VMEM((_BPW, D), jnp.float32),
            pltpu.SemaphoreType.DMA,
        ],
    )
    def _sc_gather(table_hbm, idx_hbm, out_hbm, idx_v, rows_v, sem):
        wid = lax.axis_index("s") * _SC_CORES + lax.axis_index("c")
        base = wid * _BPW
        pltpu.sync_copy(idx_hbm.at[pl.ds(base, _BPW)], idx_v)
        pltpu.async_copy(table_hbm.at[idx_v], rows_v, sem).wait()
        pltpu.sync_copy(rows_v, out_hbm.at[pl.ds(base, _BPW)])

    return _sc_gather


def kernel(inputs, embeddings, is_training):
    x = inputs.reshape(M, D)
    dist, idx3, mv3 = _dist_call()(x, embeddings)
    codebook = jnp.swapaxes(embeddings, 0, 1)
    quant = _sc_gather_call()(codebook, idx3.reshape(M))
    enc, counts = _enc_call()(idx3.reshape(M // MTE, MTE, 1))
    loss2, perp2 = _fin_call()(mv3, counts)
    return (
        quant.reshape(inputs.shape),
        loss2.reshape(()),
        perp2.reshape(()),
        enc,
        idx3.reshape(inputs.shape[:-1]),
        dist,
    )


# speedup vs baseline: 1.0631x; 1.0007x over previous
import functools

import jax
import jax.numpy as jnp
from jax import lax
from jax.experimental import pallas as pl
from jax.experimental.pallas import tpu as pltpu
from jax.experimental.pallas import tpu_sc as plsc

D = 256
K = 8192
M = 8192
COMMITMENT_COST = 0.25

MT = 2048
NT = 2048
MTE = 1024
NTE = 4096

_SC_CORES = 2
_SC_SUBCORES = 16
_NW = _SC_CORES * _SC_SUBCORES
_BPW = M // _NW

_DOT_PRECISION = lax.Precision.DEFAULT


def _dist_body(x_ref, e_ref, d_ref, idx_ref, mv_ref, minval, minidx, x2s):
    n = pl.program_id(1)
    x = x_ref[...]
    e = e_ref[...]

    @pl.when(n == 0)
    def _():
        x2s[...] = jnp.sum(x * x, axis=1, keepdims=True)

    xe = jnp.dot(x, e, preferred_element_type=jnp.float32,
                 precision=_DOT_PRECISION)
    x2 = x2s[...]
    e2 = jnp.sum(e * e, axis=0, keepdims=True)
    d = (x2 - 2.0 * xe) + e2
    d_ref[...] = d
    rmin = jnp.min(d, axis=1, keepdims=True)
    col = lax.broadcasted_iota(jnp.int32, d.shape, 1)
    ridx = jnp.min(jnp.where(d == rmin, col, jnp.int32(2**31 - 1)),
                   axis=1, keepdims=True) + n * NT

    @pl.when(n == 0)
    def _():
        minval[...] = rmin
        minidx[...] = ridx

    @pl.when(n != 0)
    def _():
        mv = minval[...]
        better = rmin < mv
        minval[...] = jnp.where(better, rmin, mv)
        minidx[...] = jnp.where(better, ridx, minidx[...])

    @pl.when(n == pl.num_programs(1) - 1)
    def _():
        idx_ref[0] = minidx[...]
        mv_ref[0] = minval[...]


@functools.cache
def _dist_call():
    return pl.pallas_call(
        _dist_body,
        grid=(M // MT, K // NT),
        in_specs=[
            pl.BlockSpec((MT, D), lambda m, n: (m, 0)),
            pl.BlockSpec((D, NT), lambda m, n: (0, n)),
        ],
        out_specs=[
            pl.BlockSpec((MT, NT), lambda m, n: (m, n)),
            pl.BlockSpec((1, MT, 1), lambda m, n: (m, 0, 0)),
            pl.BlockSpec((1, MT, 1), lambda m, n: (m, 0, 0)),
        ],
        out_shape=[
            jax.ShapeDtypeStruct((M, K), jnp.float32),
            jax.ShapeDtypeStruct((M // MT, MT, 1), jnp.int32),
            jax.ShapeDtypeStruct((M // MT, MT, 1), jnp.float32),
        ],
        scratch_shapes=[
            pltpu.VMEM((MT, 1), jnp.float32),
            pltpu.VMEM((MT, 1), jnp.int32),
            pltpu.VMEM((MT, 1), jnp.float32),
        ],
    )


def _enc_body(idx_ref, enc_ref, cnt_ref, cnt_acc):
    n = pl.program_id(0)
    m = pl.program_id(1)
    idxv = idx_ref[0]
    col = lax.broadcasted_iota(jnp.int32, (MTE, NTE), 1) + n * NTE
    enc = (col == idxv).astype(jnp.float32)
    enc_ref[...] = enc
    csum = jnp.sum(enc, axis=0, keepdims=True)

    @pl.when(m == 0)
    def _():
        cnt_acc[...] = csum

    @pl.when(m != 0)
    def _():
        cnt_acc[...] += csum

    @pl.when(m == pl.num_programs(1) - 1)
    def _():
        cnt_ref[...] = cnt_acc[...]


@functools.cache
def _enc_call():
    return pl.pallas_call(
        _enc_body,
        grid=(K // NTE, M // MTE),
        in_specs=[
            pl.BlockSpec((1, MTE, 1), lambda n, m: (m, 0, 0)),
        ],
        out_specs=[
            pl.BlockSpec((MTE, NTE), lambda n, m: (m, n)),
            pl.BlockSpec((1, NTE), lambda n, m: (0, n)),
        ],
        out_shape=[
            jax.ShapeDtypeStruct((M, K), jnp.float32),
            jax.ShapeDtypeStruct((1, K), jnp.float32),
        ],
        scratch_shapes=[
            pltpu.VMEM((1, NTE), jnp.float32),
        ],
    )


def _fin_body(mv_ref, cnt_ref, loss_ref, perp_ref):
    s = jnp.sum(mv_ref[...])
    loss_ref[0, 0] = s * ((1.0 + COMMITMENT_COST) / (M * D))
    p = cnt_ref[...] * (1.0 / M)
    h = jnp.sum(p * jnp.log(p + 1e-10))
    perp_ref[0, 0] = jnp.exp(-h)


@functools.cache
def _fin_call():
    return pl.pallas_call(
        _fin_body,
        in_specs=[
            pl.BlockSpec(memory_space=pltpu.VMEM),
            pl.BlockSpec(memory_space=pltpu.VMEM),
        ],
        out_specs=[
            pl.BlockSpec(memory_space=pltpu.SMEM),
            pl.BlockSpec(memory_space=pltpu.SMEM),
        ],
        out_shape=[
            jax.ShapeDtypeStruct((1, 1), jnp.float32),
            jax.ShapeDtypeStruct((1, 1), jnp.float32),
        ],
    )


@functools.cache
def _sc_gather_call():
    @functools.partial(
        pl.kernel,
        out_type=jax.ShapeDtypeStruct((M, D), jnp.float32),
        mesh=plsc.VectorSubcoreMesh(core_axis_name="c", subcore_axis_name="s"),
        scratch_types=[
            pltpu.VMEM((_BPW,), jnp.int32),
            pltpu.VMEM((_BPW, D), jnp.float32),
            pltpu.SemaphoreType.DMA,
        ],
    )
    def _sc_gather(table_hbm, idx_hbm, out_hbm, idx_v, rows_v, sem):
        wid = lax.axis_index("s") * _SC_CORES + lax.axis_index("c")
        base = wid * _BPW
        pltpu.sync_copy(idx_hbm.at[pl.ds(base, _BPW)], idx_v)
        pltpu.async_copy(table_hbm.at[idx_v], rows_v, sem).wait()
        pltpu.sync_copy(rows_v, out_hbm.at[pl.ds(base, _BPW)])

    return _sc_gather


def kernel(inputs, embeddings, is_training):
    x = inputs.reshape(M, D)
    dist, idx3, mv3 = _dist_call()(x, embeddings)
    codebook = jnp.swapaxes(embeddings, 0, 1)
    quant = _sc_gather_call()(codebook, idx3.reshape(M))
    enc, counts = _enc_call()(idx3.reshape(M // MTE, MTE, 1))
    loss2, perp2 = _fin_call()(mv3, counts)
    return (
        quant.reshape(inputs.shape),
        loss2.reshape(()),
        perp2.reshape(()),
        enc,
        idx3.reshape(inputs.shape[:-1]),
        dist,
    )


# speedup vs baseline: 1.0648x; 1.0016x over previous
import functools

import jax
import jax.numpy as jnp
from jax import lax
from jax.experimental import pallas as pl
from jax.experimental.pallas import tpu as pltpu
from jax.experimental.pallas import tpu_sc as plsc

D = 256
K = 8192
M = 8192
COMMITMENT_COST = 0.25

MT = 2048
NT = 2048
MTE = 2048
NTE = 2048

_SC_CORES = 2
_SC_SUBCORES = 16
_NW = _SC_CORES * _SC_SUBCORES
_BPW = M // _NW

_DOT_PRECISION = lax.Precision.DEFAULT


def _dist_body(x_ref, e_ref, d_ref, idx_ref, mv_ref, minval, minidx):
    n = pl.program_id(1)
    x = x_ref[...]
    e = e_ref[...]
    xe = jnp.dot(x, e, preferred_element_type=jnp.float32,
                 precision=_DOT_PRECISION)
    x2 = jnp.sum(x * x, axis=1, keepdims=True)
    e2 = jnp.sum(e * e, axis=0, keepdims=True)
    d = (x2 - 2.0 * xe) + e2
    d_ref[...] = d
    rmin = jnp.min(d, axis=1, keepdims=True)
    col = lax.broadcasted_iota(jnp.int32, d.shape, 1)
    ridx = jnp.min(jnp.where(d == rmin, col, jnp.int32(2**31 - 1)),
                   axis=1, keepdims=True) + n * NT

    @pl.when(n == 0)
    def _():
        minval[...] = rmin
        minidx[...] = ridx

    @pl.when(n != 0)
    def _():
        mv = minval[...]
        better = rmin < mv
        minval[...] = jnp.where(better, rmin, mv)
        minidx[...] = jnp.where(better, ridx, minidx[...])

    @pl.when(n == pl.num_programs(1) - 1)
    def _():
        idx_ref[0] = minidx[...]
        mv_ref[0] = minval[...]


@functools.cache
def _dist_call():
    return pl.pallas_call(
        _dist_body,
        grid=(M // MT, K // NT),
        in_specs=[
            pl.BlockSpec((MT, D), lambda m, n: (m, 0)),
            pl.BlockSpec((D, NT), lambda m, n: (0, n)),
        ],
        out_specs=[
            pl.BlockSpec((MT, NT), lambda m, n: (m, n)),
            pl.BlockSpec((1, MT, 1), lambda m, n: (m, 0, 0)),
            pl.BlockSpec((1, MT, 1), lambda m, n: (m, 0, 0)),
        ],
        out_shape=[
            jax.ShapeDtypeStruct((M, K), jnp.float32),
            jax.ShapeDtypeStruct((M // MT, MT, 1), jnp.int32),
            jax.ShapeDtypeStruct((M // MT, MT, 1), jnp.float32),
        ],
        scratch_shapes=[
            pltpu.VMEM((MT, 1), jnp.float32),
            pltpu.VMEM((MT, 1), jnp.int32),
        ],
    )


def _enc_body(idx_ref, enc_ref, cnt_ref, cnt_acc):
    n = pl.program_id(0)
    m = pl.program_id(1)
    idxv = idx_ref[0]
    col = lax.broadcasted_iota(jnp.int32, (MTE, NTE), 1) + n * NTE
    enc = (col == idxv).astype(jnp.float32)
    enc_ref[...] = enc
    csum = jnp.sum(enc, axis=0, keepdims=True)

    @pl.when(m == 0)
    def _():
        cnt_acc[...] = csum

    @pl.when(m != 0)
    def _():
        cnt_acc[...] += csum

    @pl.when(m == pl.num_programs(1) - 1)
    def _():
        cnt_ref[...] = cnt_acc[...]


@functools.cache
def _enc_call():
    return pl.pallas_call(
        _enc_body,
        grid=(K // NTE, M // MTE),
        in_specs=[
            pl.BlockSpec((1, MTE, 1), lambda n, m: (m, 0, 0)),
        ],
        out_specs=[
            pl.BlockSpec((MTE, NTE), lambda n, m: (m, n)),
            pl.BlockSpec((1, NTE), lambda n, m: (0, n)),
        ],
        out_shape=[
            jax.ShapeDtypeStruct((M, K), jnp.float32),
            jax.ShapeDtypeStruct((1, K), jnp.float32),
        ],
        scratch_shapes=[
            pltpu.VMEM((1, NTE), jnp.float32),
        ],
    )


def _fin_body(mv_ref, cnt_ref, loss_ref, perp_ref):
    s = jnp.sum(mv_ref[...])
    loss_ref[0, 0] = s * ((1.0 + COMMITMENT_COST) / (M * D))
    p = cnt_ref[...] * (1.0 / M)
    h = jnp.sum(p * jnp.log(p + 1e-10))
    perp_ref[0, 0] = jnp.exp(-h)


@functools.cache
def _fin_call():
    return pl.pallas_call(
        _fin_body,
        in_specs=[
            pl.BlockSpec(memory_space=pltpu.VMEM),
            pl.BlockSpec(memory_space=pltpu.VMEM),
        ],
        out_specs=[
            pl.BlockSpec(memory_space=pltpu.SMEM),
            pl.BlockSpec(memory_space=pltpu.SMEM),
        ],
        out_shape=[
            jax.ShapeDtypeStruct((1, 1), jnp.float32),
            jax.ShapeDtypeStruct((1, 1), jnp.float32),
        ],
    )


@functools.cache
def _sc_gather_call():
    @functools.partial(
        pl.kernel,
        out_type=jax.ShapeDtypeStruct((M, D), jnp.float32),
        mesh=plsc.VectorSubcoreMesh(core_axis_name="c", subcore_axis_name="s"),
        scratch_types=[
            pltpu.VMEM((_BPW,), jnp.int32),
            pltpu.VMEM((_BPW, D), jnp.float32),
            pltpu.SemaphoreType.DMA,
        ],
    )
    def _sc_gather(table_hbm, idx_hbm, out_hbm, idx_v, rows_v, sem):
        wid = lax.axis_index("s") * _SC_CORES + lax.axis_index("c")
        base = wid * _BPW
        pltpu.sync_copy(idx_hbm.at[pl.ds(base, _BPW)], idx_v)
        pltpu.async_copy(table_hbm.at[idx_v], rows_v, sem).wait()
        pltpu.sync_copy(rows_v, out_hbm.at[pl.ds(base, _BPW)])

    return _sc_gather


def kernel(inputs, embeddings, is_training):
    x = inputs.reshape(M, D)
    dist, idx3, mv3 = _dist_call()(x, embeddings)
    codebook = jnp.swapaxes(embeddings, 0, 1)
    quant = _sc_gather_call()(codebook, idx3.reshape(M))
    enc, counts = _enc_call()(idx3.reshape(M // MTE, MTE, 1))
    loss2, perp2 = _fin_call()(mv3, counts)
    return (
        quant.reshape(inputs.shape),
        loss2.reshape(()),
        perp2.reshape(()),
        enc,
        idx3.reshape(inputs.shape[:-1]),
        dist,
    )


# speedup vs baseline: 1.0963x; 1.0295x over previous
import functools

import jax
import jax.numpy as jnp
from jax import lax
from jax.experimental import pallas as pl
from jax.experimental.pallas import tpu as pltpu
from jax.experimental.pallas import tpu_sc as plsc

D = 256
K = 8192
M = 8192
COMMITMENT_COST = 0.25

MT = 2048
NT = 2048
MTE = 1024
NTE = 4096

_SC_CORES = 2
_SC_SUBCORES = 16
_NW = _SC_CORES * _SC_SUBCORES
_BPW = M // _NW

_DOT_PRECISION = lax.Precision.DEFAULT


def _dist_body(x_ref, e_ref, d_ref, idx_ref, mv_ref, minval, minidx):
    n = pl.program_id(1)
    x = x_ref[...]
    e = e_ref[...]
    xe = jnp.dot(x, e, preferred_element_type=jnp.float32,
                 precision=_DOT_PRECISION)
    x2 = jnp.sum(x * x, axis=1, keepdims=True)
    e2 = jnp.sum(e * e, axis=0, keepdims=True)
    d = (x2 - 2.0 * xe) + e2
    d_ref[...] = d
    rmin = jnp.min(d, axis=1, keepdims=True)
    col = lax.broadcasted_iota(jnp.int32, d.shape, 1)
    ridx = jnp.min(jnp.where(d == rmin, col, jnp.int32(2**31 - 1)),
                   axis=1, keepdims=True) + n * NT

    @pl.when(n == 0)
    def _():
        minval[...] = rmin
        minidx[...] = ridx

    @pl.when(n != 0)
    def _():
        mv = minval[...]
        better = rmin < mv
        minval[...] = jnp.where(better, rmin, mv)
        minidx[...] = jnp.where(better, ridx, minidx[...])

    @pl.when(n == pl.num_programs(1) - 1)
    def _():
        idx_ref[0] = minidx[...]
        mv_ref[0] = minval[...]


@functools.cache
def _dist_call():
    return pl.pallas_call(
        _dist_body,
        grid=(M // MT, K // NT),
        in_specs=[
            pl.BlockSpec((MT, D), lambda m, n: (m, 0)),
            pl.BlockSpec((D, NT), lambda m, n: (0, n)),
        ],
        out_specs=[
            pl.BlockSpec((MT, NT), lambda m, n: (m, n)),
            pl.BlockSpec((1, MT, 1), lambda m, n: (m, 0, 0)),
            pl.BlockSpec((1, MT, 1), lambda m, n: (m, 0, 0)),
        ],
        out_shape=[
            jax.ShapeDtypeStruct((M, K), jnp.float32),
            jax.ShapeDtypeStruct((M // MT, MT, 1), jnp.int32),
            jax.ShapeDtypeStruct((M // MT, MT, 1), jnp.float32),
        ],
        scratch_shapes=[
            pltpu.VMEM((MT, 1), jnp.float32),
            pltpu.VMEM((MT, 1), jnp.int32),
        ],
    )


def _enc_body(idx_ref, mv_ref, enc_ref, loss_ref, perp_ref, cnt_acc):
    n = pl.program_id(0)
    m = pl.program_id(1)
    idxv = idx_ref[0]
    col = lax.broadcasted_iota(jnp.int32, (MTE, NTE), 1) + n * NTE
    enc = (col == idxv).astype(jnp.float32)
    enc_ref[...] = enc
    csum = jnp.sum(enc, axis=0, keepdims=True)

    @pl.when(m == 0)
    def _():
        cnt_acc[:, pl.ds(n * NTE, NTE)] = csum

    @pl.when(m != 0)
    def _():
        cnt_acc[:, pl.ds(n * NTE, NTE)] += csum

    @pl.when((n == pl.num_programs(0) - 1) & (m == pl.num_programs(1) - 1))
    def _():
        s = jnp.sum(mv_ref[...])
        loss_ref[0, 0] = s * ((1.0 + COMMITMENT_COST) / (M * D))
        p = cnt_acc[...] * (1.0 / M)
        h = jnp.sum(p * jnp.log(p + 1e-10))
        perp_ref[0, 0] = jnp.exp(-h)


@functools.cache
def _enc_call():
    return pl.pallas_call(
        _enc_body,
        grid=(K // NTE, M // MTE),
        in_specs=[
            pl.BlockSpec((1, MTE, 1), lambda n, m: (m, 0, 0)),
            pl.BlockSpec((M // MT, MT, 1), lambda n, m: (0, 0, 0)),
        ],
        out_specs=[
            pl.BlockSpec((MTE, NTE), lambda n, m: (m, n)),
            pl.BlockSpec(memory_space=pltpu.SMEM),
            pl.BlockSpec(memory_space=pltpu.SMEM),
        ],
        out_shape=[
            jax.ShapeDtypeStruct((M, K), jnp.float32),
            jax.ShapeDtypeStruct((1, 1), jnp.float32),
            jax.ShapeDtypeStruct((1, 1), jnp.float32),
        ],
        scratch_shapes=[
            pltpu.VMEM((1, K), jnp.float32),
        ],
    )


def _fin_body(mv_ref, cnt_ref, loss_ref, perp_ref):
    s = jnp.sum(mv_ref[...])
    loss_ref[0, 0] = s * ((1.0 + COMMITMENT_COST) / (M * D))
    p = cnt_ref[...] * (1.0 / M)
    h = jnp.sum(p * jnp.log(p + 1e-10))
    perp_ref[0, 0] = jnp.exp(-h)


@functools.cache
def _fin_call():
    return pl.pallas_call(
        _fin_body,
        in_specs=[
            pl.BlockSpec(memory_space=pltpu.VMEM),
            pl.BlockSpec(memory_space=pltpu.VMEM),
        ],
        out_specs=[
            pl.BlockSpec(memory_space=pltpu.SMEM),
            pl.BlockSpec(memory_space=pltpu.SMEM),
        ],
        out_shape=[
            jax.ShapeDtypeStruct((1, 1), jnp.float32),
            jax.ShapeDtypeStruct((1, 1), jnp.float32),
        ],
    )


@functools.cache
def _sc_gather_call():
    @functools.partial(
        pl.kernel,
        out_type=jax.ShapeDtypeStruct((M, D), jnp.float32),
        mesh=plsc.VectorSubcoreMesh(core_axis_name="c", subcore_axis_name="s"),
        scratch_types=[
            pltpu.VMEM((_BPW,), jnp.int32),
            pltpu.VMEM((_BPW, D), jnp.float32),
            pltpu.SemaphoreType.DMA,
        ],
    )
    def _sc_gather(table_hbm, idx_hbm, out_hbm, idx_v, rows_v, sem):
        wid = lax.axis_index("s") * _SC_CORES + lax.axis_index("c")
        base = wid * _BPW
        pltpu.sync_copy(idx_hbm.at[pl.ds(base, _BPW)], idx_v)
        pltpu.async_copy(table_hbm.at[idx_v], rows_v, sem).wait()
        pltpu.sync_copy(rows_v, out_hbm.at[pl.ds(base, _BPW)])

    return _sc_gather


def kernel(inputs, embeddings, is_training):
    x = inputs.reshape(M, D)
    dist, idx3, mv3 = _dist_call()(x, embeddings)
    codebook = jnp.swapaxes(embeddings, 0, 1)
    quant = _sc_gather_call()(codebook, idx3.reshape(M))
    enc, loss2, perp2 = _enc_call()(idx3.reshape(M // MTE, MTE, 1), mv3)
    return (
        quant.reshape(inputs.shape),
        loss2.reshape(()),
        perp2.reshape(()),
        enc,
        idx3.reshape(inputs.shape[:-1]),
        dist,
    )


# speedup vs baseline: 1.0971x; 1.0007x over previous
import functools

import jax
import jax.numpy as jnp
from jax import lax
from jax.experimental import pallas as pl
from jax.experimental.pallas import tpu as pltpu
from jax.experimental.pallas import tpu_sc as plsc

D = 256
K = 8192
M = 8192
COMMITMENT_COST = 0.25

MT = 2048
NT = 2048
MTE = 1024
NTE = 4096

_SC_CORES = 2
_SC_SUBCORES = 16
_NW = _SC_CORES * _SC_SUBCORES
_BPW = M // _NW

_DOT_PRECISION = lax.Precision.DEFAULT


def _dist_body(x_ref, e_ref, d_ref, idx_ref, mv_ref, minval, minidx):
    n = pl.program_id(1)
    x = x_ref[...]
    e = e_ref[...]
    xe = jnp.dot(x, e, preferred_element_type=jnp.float32,
                 precision=_DOT_PRECISION)
    x2 = jnp.sum(x * x, axis=1, keepdims=True)
    e2 = jnp.sum(e * e, axis=0, keepdims=True)
    d = (x2 - 2.0 * xe) + e2
    d_ref[...] = d
    rmin = jnp.min(d, axis=1, keepdims=True)
    col = lax.broadcasted_iota(jnp.int32, d.shape, 1)
    ridx = jnp.min(jnp.where(d == rmin, col, jnp.int32(2**31 - 1)),
                   axis=1, keepdims=True) + n * NT

    @pl.when(n == 0)
    def _():
        minval[...] = rmin
        minidx[...] = ridx

    @pl.when(n != 0)
    def _():
        mv = minval[...]
        better = rmin < mv
        minval[...] = jnp.where(better, rmin, mv)
        minidx[...] = jnp.where(better, ridx, minidx[...])

    @pl.when(n == pl.num_programs(1) - 1)
    def _():
        idx_ref[0] = minidx[...]
        mv_ref[0] = minval[...]


@functools.cache
def _dist_call():
    return pl.pallas_call(
        _dist_body,
        grid=(M // MT, K // NT),
        in_specs=[
            pl.BlockSpec((MT, D), lambda m, n: (m, 0)),
            pl.BlockSpec((D, NT), lambda m, n: (0, n)),
        ],
        out_specs=[
            pl.BlockSpec((MT, NT), lambda m, n: (m, n)),
            pl.BlockSpec((1, MT, 1), lambda m, n: (m, 0, 0)),
            pl.BlockSpec((1, MT, 1), lambda m, n: (m, 0, 0)),
        ],
        out_shape=[
            jax.ShapeDtypeStruct((M, K), jnp.float32),
            jax.ShapeDtypeStruct((M // MT, MT, 1), jnp.int32),
            jax.ShapeDtypeStruct((M // MT, MT, 1), jnp.float32),
        ],
        scratch_shapes=[
            pltpu.VMEM((MT, 1), jnp.float32),
            pltpu.VMEM((MT, 1), jnp.int32),
        ],
    )


def _enc_body(idx_ref, mv_ref, enc_ref, loss_ref, perp_ref, cnt_acc):
    n = pl.program_id(0)
    m = pl.program_id(1)
    idxv = idx_ref[0]
    col = lax.broadcasted_iota(jnp.int32, (MTE, NTE), 1) + n * NTE
    enc = (col == idxv).astype(jnp.float32)
    enc_ref[...] = enc
    csum = jnp.sum(enc, axis=0, keepdims=True)

    @pl.when(m == 0)
    def _():
        cnt_acc[:, pl.ds(n * NTE, NTE)] = csum

    @pl.when(m != 0)
    def _():
        cnt_acc[:, pl.ds(n * NTE, NTE)] += csum

    @pl.when((n == pl.num_programs(0) - 1) & (m == pl.num_programs(1) - 1))
    def _():
        s = jnp.sum(mv_ref[...])
        loss_ref[0, 0] = s * ((1.0 + COMMITMENT_COST) / (M * D))
        p = cnt_acc[...] * (1.0 / M)
        h = jnp.sum(p * jnp.log(p + 1e-10))
        perp_ref[0, 0] = jnp.exp(-h)


@functools.cache
def _enc_call():
    return pl.pallas_call(
        _enc_body,
        grid=(K // NTE, M // MTE),
        in_specs=[
            pl.BlockSpec((1, MTE, 1), lambda n, m: (m, 0, 0)),
            pl.BlockSpec((M // MT, MT, 1), lambda n, m: (0, 0, 0)),
        ],
        out_specs=[
            pl.BlockSpec((MTE, NTE), lambda n, m: (m, n)),
            pl.BlockSpec(memory_space=pltpu.SMEM),
            pl.BlockSpec(memory_space=pltpu.SMEM),
        ],
        out_shape=[
            jax.ShapeDtypeStruct((M, K), jnp.float32),
            jax.ShapeDtypeStruct((1, 1), jnp.float32),
            jax.ShapeDtypeStruct((1, 1), jnp.float32),
        ],
        scratch_shapes=[
            pltpu.VMEM((1, K), jnp.float32),
        ],
    )


@functools.cache
def _sc_gather_call():
    @functools.partial(
        pl.kernel,
        out_type=jax.ShapeDtypeStruct((M, D), jnp.float32),
        mesh=plsc.VectorSubcoreMesh(core_axis_name="c", subcore_axis_name="s"),
        scratch_types=[
            pltpu.VMEM((_BPW,), jnp.int32),
            pltpu.VMEM((_BPW, D), jnp.float32),
            pltpu.SemaphoreType.DMA,
        ],
    )
    def _sc_gather(table_hbm, idx_hbm, out_hbm, idx_v, rows_v, sem):
        wid = lax.axis_index("s") * _SC_CORES + lax.axis_index("c")
        base = wid * _BPW
        pltpu.sync_copy(idx_hbm.at[pl.ds(base, _BPW)], idx_v)
        pltpu.async_copy(table_hbm.at[idx_v], rows_v, sem).wait()
        pltpu.sync_copy(rows_v, out_hbm.at[pl.ds(base, _BPW)])

    return _sc_gather


def kernel(inputs, embeddings, is_training):
    x = inputs.reshape(M, D)
    dist, idx3, mv3 = _dist_call()(x, embeddings)
    codebook = jnp.swapaxes(embeddings, 0, 1)
    quant = _sc_gather_call()(codebook, idx3.reshape(M))
    enc, loss2, perp2 = _enc_call()(idx3.reshape(M // MTE, MTE, 1), mv3)
    return (
        quant.reshape(inputs.shape),
        loss2.reshape(()),
        perp2.reshape(()),
        enc,
        idx3.reshape(inputs.shape[:-1]),
        dist,
    )


# speedup vs baseline: 1.0972x; 1.0001x over previous
import functools

import jax
import jax.numpy as jnp
from jax import lax
from jax.experimental import pallas as pl
from jax.experimental.pallas import tpu as pltpu
from jax.experimental.pallas import tpu_sc as plsc

D = 256
K = 8192
M = 8192
COMMITMENT_COST = 0.25

MT = 2048
NT = 2048
MTE = 1024
NTE = 4096

_SC_CORES = 2
_SC_SUBCORES = 16
_NW = _SC_CORES * _SC_SUBCORES
_BPW = M // _NW

_DOT_PRECISION = lax.Precision.DEFAULT


def _dist_body(x_ref, e_ref, d_ref, idx_ref, mv_ref, minval, minidx):
    n = pl.program_id(1)
    x = x_ref[...]
    e = e_ref[:, pl.ds(n * NT, NT)]
    xe = jnp.dot(x, e, preferred_element_type=jnp.float32,
                 precision=_DOT_PRECISION)
    x2 = jnp.sum(x * x, axis=1, keepdims=True)
    e2 = jnp.sum(e * e, axis=0, keepdims=True)
    d = (x2 - 2.0 * xe) + e2
    d_ref[...] = d
    rmin = jnp.min(d, axis=1, keepdims=True)
    col = lax.broadcasted_iota(jnp.int32, d.shape, 1)
    ridx = jnp.min(jnp.where(d == rmin, col, jnp.int32(2**31 - 1)),
                   axis=1, keepdims=True) + n * NT

    @pl.when(n == 0)
    def _():
        minval[...] = rmin
        minidx[...] = ridx

    @pl.when(n != 0)
    def _():
        mv = minval[...]
        better = rmin < mv
        minval[...] = jnp.where(better, rmin, mv)
        minidx[...] = jnp.where(better, ridx, minidx[...])

    @pl.when(n == pl.num_programs(1) - 1)
    def _():
        idx_ref[0] = minidx[...]
        mv_ref[0] = minval[...]


@functools.cache
def _dist_call():
    return pl.pallas_call(
        _dist_body,
        grid=(M // MT, K // NT),
        in_specs=[
            pl.BlockSpec((MT, D), lambda m, n: (m, 0)),
            pl.BlockSpec((D, K), lambda m, n: (0, 0)),
        ],
        out_specs=[
            pl.BlockSpec((MT, NT), lambda m, n: (m, n)),
            pl.BlockSpec((1, MT, 1), lambda m, n: (m, 0, 0)),
            pl.BlockSpec((1, MT, 1), lambda m, n: (m, 0, 0)),
        ],
        out_shape=[
            jax.ShapeDtypeStruct((M, K), jnp.float32),
            jax.ShapeDtypeStruct((M // MT, MT, 1), jnp.int32),
            jax.ShapeDtypeStruct((M // MT, MT, 1), jnp.float32),
        ],
        scratch_shapes=[
            pltpu.VMEM((MT, 1), jnp.float32),
            pltpu.VMEM((MT, 1), jnp.int32),
        ],
    )


def _enc_body(idx_ref, mv_ref, enc_ref, loss_ref, perp_ref, cnt_acc):
    n = pl.program_id(0)
    m = pl.program_id(1)
    idxv = idx_ref[0]
    col = lax.broadcasted_iota(jnp.int32, (MTE, NTE), 1) + n * NTE
    enc = (col == idxv).astype(jnp.float32)
    enc_ref[...] = enc
    csum = jnp.sum(enc, axis=0, keepdims=True)

    @pl.when(m == 0)
    def _():
        cnt_acc[:, pl.ds(n * NTE, NTE)] = csum

    @pl.when(m != 0)
    def _():
        cnt_acc[:, pl.ds(n * NTE, NTE)] += csum

    @pl.when((n == pl.num_programs(0) - 1) & (m == pl.num_programs(1) - 1))
    def _():
        s = jnp.sum(mv_ref[...])
        loss_ref[0, 0] = s * ((1.0 + COMMITMENT_COST) / (M * D))
        p = cnt_acc[...] * (1.0 / M)
        h = jnp.sum(p * jnp.log(p + 1e-10))
        perp_ref[0, 0] = jnp.exp(-h)


@functools.cache
def _enc_call():
    return pl.pallas_call(
        _enc_body,
        grid=(K // NTE, M // MTE),
        in_specs=[
            pl.BlockSpec((1, MTE, 1), lambda n, m: (m, 0, 0)),
            pl.BlockSpec((M // MT, MT, 1), lambda n, m: (0, 0, 0)),
        ],
        out_specs=[
            pl.BlockSpec((MTE, NTE), lambda n, m: (m, n)),
            pl.BlockSpec(memory_space=pltpu.SMEM),
            pl.BlockSpec(memory_space=pltpu.SMEM),
        ],
        out_shape=[
            jax.ShapeDtypeStruct((M, K), jnp.float32),
            jax.ShapeDtypeStruct((1, 1), jnp.float32),
            jax.ShapeDtypeStruct((1, 1), jnp.float32),
        ],
        scratch_shapes=[
            pltpu.VMEM((1, K), jnp.float32),
        ],
    )


@functools.cache
def _sc_gather_call():
    @functools.partial(
        pl.kernel,
        out_type=jax.ShapeDtypeStruct((M, D), jnp.float32),
        mesh=plsc.VectorSubcoreMesh(core_axis_name="c", subcore_axis_name="s"),
        scratch_types=[
            pltpu.VMEM((_BPW,), jnp.int32),
            pltpu.VMEM((_BPW, D), jnp.float32),
            pltpu.SemaphoreType.DMA,
        ],
    )
    def _sc_gather(table_hbm, idx_hbm, out_hbm, idx_v, rows_v, sem):
        wid = lax.axis_index("s") * _SC_CORES + lax.axis_index("c")
        base = wid * _BPW
        pltpu.sync_copy(idx_hbm.at[pl.ds(base, _BPW)], idx_v)
        pltpu.async_copy(table_hbm.at[idx_v], rows_v, sem).wait()
        pltpu.sync_copy(rows_v, out_hbm.at[pl.ds(base, _BPW)])

    return _sc_gather


def kernel(inputs, embeddings, is_training):
    x = inputs.reshape(M, D)
    dist, idx3, mv3 = _dist_call()(x, embeddings)
    codebook = jnp.swapaxes(embeddings, 0, 1)
    quant = _sc_gather_call()(codebook, idx3.reshape(M))
    enc, loss2, perp2 = _enc_call()(idx3.reshape(M // MTE, MTE, 1), mv3)
    return (
        quant.reshape(inputs.shape),
        loss2.reshape(()),
        perp2.reshape(()),
        enc,
        idx3.reshape(inputs.shape[:-1]),
        dist,
    )
